# baseline (speedup 1.0000x reference)
import functools

import jax
import jax.numpy as jnp
from jax import lax
from jax.experimental import pallas as pl
from jax.experimental.pallas import tpu as pltpu
from jax.experimental.pallas import tpu_sc as plsc

_STATIC_LEN = 8192
_HIDDEN = 1024
_NUM_CORES = 2
_NUM_SUBCORES = 16
_NUM_WORKERS = _NUM_CORES * _NUM_SUBCORES
_ROWS_PER_WORKER = _STATIC_LEN // _NUM_WORKERS
_CHUNK = 32
_NCHUNKS = _ROWS_PER_WORKER // _CHUNK


def _sc_slice_copy(src_hbm, start_hbm, out_hbm,
                   start_v, buf0, buf1, ls0, ls1, ss0, ss1):
    wid = lax.axis_index("s") * _NUM_CORES + lax.axis_index("c")
    pltpu.sync_copy(start_hbm, start_v)
    start = pl.multiple_of(start_v[...][0], 8)

    base = wid * _ROWS_PER_WORKER
    bufs = (buf0, buf1)
    lsems = (ls0, ls1)
    ssems = (ss0, ss1)

    def load(g):
        return pltpu.async_copy(
            src_hbm.at[pl.ds(start + base + g * _CHUNK, _CHUNK), :],
            bufs[g % 2], lsems[g % 2])

    def store(g):
        return pltpu.async_copy(
            bufs[g % 2],
            out_hbm.at[pl.ds(base + g * _CHUNK, _CHUNK), :],
            ssems[g % 2])

    loads = [None] * _NCHUNKS
    stores = [None] * _NCHUNKS
    loads[0] = load(0)
    for g in range(_NCHUNKS):
        if g + 1 < _NCHUNKS:
            if g >= 1:
                stores[g - 1].wait()
            loads[g + 1] = load(g + 1)
        loads[g].wait()
        stores[g] = store(g)
    stores[_NCHUNKS - 2].wait()
    stores[_NCHUNKS - 1].wait()


@jax.jit
def kernel(freqs, seq_len):
    src = freqs.reshape(_STATIC_LEN * 8, _HIDDEN)
    start = (jnp.asarray(seq_len, jnp.int32) - _STATIC_LEN)
    start_vec = jnp.full((16,), start, dtype=jnp.int32)

    mesh = plsc.VectorSubcoreMesh(
        core_axis_name="c", subcore_axis_name="s",
        num_cores=_NUM_CORES, num_subcores=_NUM_SUBCORES)
    out = pl.kernel(
        _sc_slice_copy,
        out_type=jax.ShapeDtypeStruct((_STATIC_LEN, _HIDDEN), jnp.float32),
        mesh=mesh,
        scratch_types=[
            pltpu.VMEM((16,), jnp.int32),
            pltpu.VMEM((_CHUNK, _HIDDEN), jnp.float32),
            pltpu.VMEM((_CHUNK, _HIDDEN), jnp.float32),
            pltpu.SemaphoreType.DMA,
            pltpu.SemaphoreType.DMA,
            pltpu.SemaphoreType.DMA,
            pltpu.SemaphoreType.DMA,
        ],
    )(src, start_vec)
    return out.reshape(1, _STATIC_LEN, _HIDDEN)
